# baseline (speedup 1.0000x reference)
import jax
import jax.numpy as jnp
from jax import lax
from jax.experimental import pallas as pl
from jax.experimental.pallas import tpu as pltpu
from jax.experimental.pallas import tpu_sc as plsc

N = 10000
E = 320000
IN_CH = 128
HID = 64
HEADS = 6
OUT_CH = 32
H1 = HEADS * HID
AW2 = OUT_CH + 16
K = 80
G = 5
NCH = E // K
BLK = 1000

_mesh = plsc.VectorSubcoreMesh(core_axis_name="c", subcore_axis_name="s")
_sc_params = pltpu.CompilerParams(use_tc_tiling_on_sc=False)


def _dense1_body(x_ref, w1_ref, was_ref, wad_ref, h_ref, as_ref, ad_ref):
    j = pl.program_id(1)
    h = jnp.dot(x_ref[...], w1_ref[...], preferred_element_type=jnp.float32)
    h_ref[...] = h
    pas = jnp.dot(h, was_ref[...], preferred_element_type=jnp.float32)
    pad = jnp.dot(h, wad_ref[...], preferred_element_type=jnp.float32)

    @pl.when(j == 0)
    def _():
        as_ref[...] = pas
        ad_ref[...] = pad

    @pl.when(j > 0)
    def _():
        as_ref[...] += pas
        ad_ref[...] += pad


def _dense2_body(acca_ref, accb_ref, c45a_ref, c45b_ref, sa_ref, sb_ref,
                 b1_ref, w2_ref, was2_ref, wad2_ref, p01_ref, p23_ref,
                 p45_ref, h2_ref, ad2_ref):
    rec = 1.0 / (sa_ref[...] + sb_ref[...] + 1e-16)
    h1 = jnp.concatenate([
        acca_ref[...] * jnp.dot(rec, p01_ref[...],
                                preferred_element_type=jnp.float32),
        accb_ref[...] * jnp.dot(rec, p23_ref[...],
                                preferred_element_type=jnp.float32),
        (c45a_ref[...] + c45b_ref[...]) * jnp.dot(
            rec, p45_ref[...], preferred_element_type=jnp.float32),
    ], axis=1)
    h1 = jnp.maximum(h1 + b1_ref[...], 0.0)
    h2 = jnp.dot(h1, w2_ref[...], preferred_element_type=jnp.float32)
    as2 = jnp.dot(h2, was2_ref[...], preferred_element_type=jnp.float32)
    h2_ref[...] = jnp.concatenate([h2, as2], axis=1)
    ad2_ref[...] = jnp.dot(h2, wad2_ref[...],
                           preferred_element_type=jnp.float32)


def _final_body(acca_ref, accb_ref, b2_ref, o_ref):
    acca = acca_ref[...]
    accb = accb_ref[...]
    num = acca[:, :OUT_CH] + accb[:, :OUT_CH]
    srow = acca[:, OUT_CH:] + accb[:, OUT_CH:]
    lane = lax.broadcasted_iota(jnp.int32, srow.shape, 1)
    s = jnp.sum(jnp.where(lane == 0, srow, 0.0), axis=1, keepdims=True)
    h2 = num / (s + 1e-16) + b2_ref[...]
    m = jnp.max(h2, axis=1, keepdims=True)
    z = h2 - m
    o_ref[...] = z - jnp.log(jnp.sum(jnp.exp(z), axis=1, keepdims=True))


def _zero_shared(z_hbm, acc_sh, t):
    zb = t * 624
    pltpu.sync_copy(z_hbm.at[pl.ds(zb, 624)], acc_sh.at[pl.ds(zb, 624)])

    @pl.when(t == 15)
    def _():
        pltpu.sync_copy(z_hbm.at[pl.ds(9984, 16)], acc_sh.at[pl.ds(9984, 16)])


def _copy_out(acc_sh, out_hbm, c, t):
    zb = t * 624
    pltpu.sync_copy(acc_sh.at[pl.ds(zb, 624)],
                    out_hbm.at[pl.ds(c * N + zb, 624)])

    @pl.when(t == 15)
    def _():
        pltpu.sync_copy(acc_sh.at[pl.ds(9984, 16)],
                        out_hbm.at[pl.ds(c * N + 9984, 16)])


def _shift_idx(grp, delta, out=None):
    dst = grp if out is None else out
    for r in range(grp.shape[0]):
        for s5 in range(K // 16):
            sl = pl.ds(s5 * 16, 16)
            dst[r, sl] = grp[r, sl] + delta


def _make_ps1():
    def body(src2, dst2, astab, adtab, h_all, zs_hbm, z128_hbm,
             p3, s_out, a45_out,
             sgrp, sg2, dgrp, pbuf, a0, a1, b0, b1, h0, h1, s_sh, a45_sh,
             sem_s, sem_d, sem_a0, sem_a1, sem_b0, sem_b1, sem_h0, sem_h1,
             sem_x0, sem_x1, sem_y0, sem_y1):
        c = lax.axis_index("c")
        t = lax.axis_index("s")
        _zero_shared(zs_hbm, s_sh, t)
        _zero_shared(z128_hbm, a45_sh, t)
        plsc.subcore_barrier()
        abufs = (a0, a1)
        bbufs = (b0, b1)
        hbufs = (h0, h1)
        asems = (sem_a0, sem_a1)
        bsems = (sem_b0, sem_b1)
        hsems = (sem_h0, sem_h1)
        xsems = (sem_x0, sem_x1)
        ysems = (sem_y0, sem_y1)
        rbase = c * (NCH // 2) + t * (NCH // 32)
        two_n = jnp.int32(2 * N)

        def group(g, carry):
            r0 = rbase + g * G
            cps = pltpu.async_copy(src2.at[pl.ds(r0, G)], sgrp, sem_s)
            cpd = pltpu.async_copy(dst2.at[pl.ds(r0, G)], dgrp, sem_d)
            cps.wait()
            cpd.wait()
            _shift_idx(sgrp, two_n, out=sg2)
            cpa = [None] * G
            cpb = [None] * G
            cph = [None] * G
            scx = [None] * G
            scy = [None] * G
            cpa[0] = pltpu.async_copy(astab.at[sgrp.at[0]], a0, sem_a0)
            cpb[0] = pltpu.async_copy(adtab.at[dgrp.at[0]], b0, sem_b0)
            cph[0] = pltpu.async_copy(h_all.at[sg2.at[0]], h0, sem_h0)
            for b in range(G):
                if b >= 1:
                    scx[b - 1].wait()
                    scy[b - 1].wait()
                if b + 1 < G:
                    nb = (b + 1) % 2
                    cpa[b + 1] = pltpu.async_copy(
                        astab.at[sgrp.at[b + 1]], abufs[nb], asems[nb])
                    cpb[b + 1] = pltpu.async_copy(
                        adtab.at[dgrp.at[b + 1]], bbufs[nb], bsems[nb])
                    cph[b + 1] = pltpu.async_copy(
                        h_all.at[sg2.at[b + 1]], hbufs[nb], hsems[nb])
                cpa[b].wait()
                cpb[b].wait()
                cph[b].wait()
                asr_v = abufs[b % 2]
                adr_v = bbufs[b % 2]
                h_v = hbufs[b % 2]

                @plsc.parallel_loop(0, K, unroll=2)
                def edge(k):
                    e = asr_v[k] + adr_v[k]
                    p = jnp.exp(jnp.maximum(e, 0.2 * e))
                    pbuf[b, k] = p
                    for j in range(2):
                        pj = jnp.take_along_axis(
                            p, jnp.full((16,), 4 + j, jnp.int32), axis=0,
                            mode="promise_in_bounds")
                        for v in range(4):
                            sl = pl.ds(j * 64 + v * 16, 16)
                            h_v[k, sl] = h_v[k, sl] * pj

                scx[b] = pltpu.async_copy(
                    pbuf.at[b], s_sh.at[dgrp.at[b]], xsems[b % 2], add=True)
                scy[b] = pltpu.async_copy(
                    h_v, a45_sh.at[dgrp.at[b]], ysems[b % 2], add=True)
            scx[G - 1].wait()
            scy[G - 1].wait()
            pltpu.sync_copy(pbuf, p3.at[pl.ds(r0, G)])
            return carry

        lax.fori_loop(0, NCH // 32 // G, group, 0)
        plsc.subcore_barrier()
        _copy_out(s_sh, s_out, c, t)
        _copy_out(a45_sh, a45_out, c, t)

    return pl.kernel(
        body,
        out_type=[
            jax.ShapeDtypeStruct((NCH, K, 16), jnp.float32),
            jax.ShapeDtypeStruct((2 * N, 16), jnp.float32),
            jax.ShapeDtypeStruct((2 * N, 128), jnp.float32),
        ],
        mesh=_mesh,
        compiler_params=_sc_params,
        scratch_types=[
            pltpu.VMEM((G, K), jnp.int32),
            pltpu.VMEM((G, K), jnp.int32),
            pltpu.VMEM((G, K), jnp.int32),
            pltpu.VMEM((G, K, 16), jnp.float32),
            pltpu.VMEM((K, 16), jnp.float32),
            pltpu.VMEM((K, 16), jnp.float32),
            pltpu.VMEM((K, 16), jnp.float32),
            pltpu.VMEM((K, 16), jnp.float32),
            pltpu.VMEM((K, 128), jnp.float32),
            pltpu.VMEM((K, 128), jnp.float32),
            pltpu.VMEM_SHARED((N, 16), jnp.float32),
            pltpu.VMEM_SHARED((N, 128), jnp.float32),
        ] + [pltpu.SemaphoreType.DMA] * 12,
    )


GM = 5


def _make_m1():
    def body(src2, dst2, p3, h_all, z128_hbm, out_hbm,
             sa, da, pa, sb, db, pb, h0, h1,
             sem_sa, sem_da, sem_pa, sem_sb, sem_db, sem_pb,
             sem_h0, sem_h1, sem_x0, sem_x1, acc_sh):
        c = lax.axis_index("c")
        t = lax.axis_index("s")
        hoff = c * 2
        _zero_shared(z128_hbm, acc_sh, t)
        plsc.subcore_barrier()
        hbufs = (h0, h1)
        hsems = (sem_h0, sem_h1)
        ssems = (sem_x0, sem_x1)
        rbase = t * (NCH // 16)
        c_n = c * jnp.int32(N)
        npairs = NCH // 16 // (2 * GM)

        def issue_idx(r0, sg, dg, pg, ss, sd, sp):
            return (pltpu.async_copy(src2.at[pl.ds(r0, GM)], sg, ss),
                    pltpu.async_copy(dst2.at[pl.ds(r0, GM)], dg, sd),
                    pltpu.async_copy(p3.at[pl.ds(r0, GM)], pg, sp))

        issue_idx(rbase, sa, da, pa, sem_sa, sem_da, sem_pa)

        def pair(i, carry):
            r0 = rbase + i * 2 * GM
            cpsB = issue_idx(r0 + GM, sb, db, pb, sem_sb, sem_db, sem_pb)

            def phase(cps, r0p, sg, dg, pg, next_r0, guard):
                if cps is None:
                    pltpu.make_async_copy(
                        src2.at[pl.ds(r0p, GM)], sg, sem_sa).wait()
                    pltpu.make_async_copy(
                        dst2.at[pl.ds(r0p, GM)], dg, sem_da).wait()
                    pltpu.make_async_copy(
                        p3.at[pl.ds(r0p, GM)], pg, sem_pa).wait()
                else:
                    for q in cps:
                        q.wait()
                _shift_idx(sg, c_n)
                cph = [None] * GM
                scp = [None] * GM
                cph[0] = pltpu.async_copy(h_all.at[sg.at[0]], hbufs[0],
                                          hsems[0])
                for b in range(GM):
                    if b >= 1:
                        scp[b - 1].wait()
                    if b + 1 < GM:
                        nb = (b + 1) % 2
                        cph[b + 1] = pltpu.async_copy(
                            h_all.at[sg.at[b + 1]], hbufs[nb], hsems[nb])
                    if b == GM - 1 and next_r0 is not None:
                        @pl.when(guard)
                        def _():
                            issue_idx(next_r0, sa, da, pa,
                                      sem_sa, sem_da, sem_pa)
                    cph[b].wait()
                    h_v = hbufs[b % 2]

                    @plsc.parallel_loop(0, K, unroll=2)
                    def edge(k):
                        pvec = pg[b, k]
                        for j in range(2):
                            pj = jnp.take_along_axis(
                                pvec, jnp.full((16,), hoff + j, jnp.int32),
                                axis=0, mode="promise_in_bounds")
                            for v in range(4):
                                sl = pl.ds(j * 64 + v * 16, 16)
                                h_v[k, sl] = h_v[k, sl] * pj

                    scp[b] = pltpu.async_copy(
                        h_v, acc_sh.at[dg.at[b]], ssems[b % 2], add=True)
                scp[GM - 1].wait()

            phase(None, r0, sa, da, pa, r0 + 2 * GM, i + 1 < npairs)
            phase(cpsB, r0 + GM, sb, db, pb, None, None)
            return carry

        lax.fori_loop(0, npairs, pair, 0)
        plsc.subcore_barrier()
        _copy_out(acc_sh, out_hbm, c, t)

    return pl.kernel(
        body,
        out_type=jax.ShapeDtypeStruct((2 * N, 128), jnp.float32),
        mesh=_mesh,
        compiler_params=_sc_params,
        scratch_types=[
            pltpu.VMEM((GM, K), jnp.int32),
            pltpu.VMEM((GM, K), jnp.int32),
            pltpu.VMEM((GM, K, 16), jnp.float32),
            pltpu.VMEM((GM, K), jnp.int32),
            pltpu.VMEM((GM, K), jnp.int32),
            pltpu.VMEM((GM, K, 16), jnp.float32),
            pltpu.VMEM((K, 128), jnp.float32),
            pltpu.VMEM((K, 128), jnp.float32),
        ] + [pltpu.SemaphoreType.DMA] * 10 + [
            pltpu.VMEM_SHARED((N, 128), jnp.float32),
        ],
    )


GE = 25


def _make_edge2():
    def body(src2, dst2, adtab, h2_hbm, z48_hbm, out_hbm,
             sgrp, dgrp, b0, b1, h0, h1, m0, m1b,
             sem_s, sem_d, sem_b0, sem_b1, sem_h0, sem_h1, sem_x0, sem_x1,
             acc_sh):
        c = lax.axis_index("c")
        t = lax.axis_index("s")
        _zero_shared(z48_hbm, acc_sh, t)
        plsc.subcore_barrier()
        bbufs = (b0, b1)
        hbufs = (h0, h1)
        bsems = (sem_b0, sem_b1)
        hsems = (sem_h0, sem_h1)
        msgbufs = (m0, m1b)
        xsems = (sem_x0, sem_x1)
        rbase = c * (NCH // 2) + t * (NCH // 32)
        zeros16 = jnp.zeros((16,), jnp.int32)

        def group(g, carry):
            r0 = rbase + g * GE
            cps = pltpu.async_copy(src2.at[pl.ds(r0, GE)], sgrp, sem_s)
            cpd = pltpu.async_copy(dst2.at[pl.ds(r0, GE)], dgrp, sem_d)
            cps.wait()
            cpd.wait()
            cpb = [None] * GE
            cph = [None] * GE
            scp = [None] * GE
            cpb[0] = pltpu.async_copy(adtab.at[dgrp.at[0]], b0, sem_b0)
            cph[0] = pltpu.async_copy(h2_hbm.at[sgrp.at[0]], h0, sem_h0)
            for b in range(GE):
                if b >= 2:
                    scp[b - 2].wait()
                if b + 1 < GE:
                    nb = (b + 1) % 2
                    cpb[b + 1] = pltpu.async_copy(
                        adtab.at[dgrp.at[b + 1]], bbufs[nb], bsems[nb])
                    cph[b + 1] = pltpu.async_copy(
                        h2_hbm.at[sgrp.at[b + 1]], hbufs[nb], hsems[nb])
                cpb[b].wait()
                cph[b].wait()
                adr_v = bbufs[b % 2]
                h_v = hbufs[b % 2]
                msg_v = msgbufs[b % 2]

                @plsc.parallel_loop(0, K, unroll=4)
                def edge(k):
                    e = h_v[k, pl.ds(OUT_CH, 16)] + adr_v[k]
                    p = jnp.exp(jnp.maximum(e, 0.2 * e))
                    msg_v[k, pl.ds(OUT_CH, 16)] = p
                    p0 = jnp.take_along_axis(p, zeros16, axis=0,
                                             mode="promise_in_bounds")
                    msg_v[k, pl.ds(0, 16)] = h_v[k, pl.ds(0, 16)] * p0
                    msg_v[k, pl.ds(16, 16)] = h_v[k, pl.ds(16, 16)] * p0

                scp[b] = pltpu.async_copy(
                    msg_v, acc_sh.at[dgrp.at[b]], xsems[b % 2], add=True)
            scp[GE - 2].wait()
            scp[GE - 1].wait()
            return carry

        lax.fori_loop(0, NCH // 32 // GE, group, 0)
        plsc.subcore_barrier()
        _copy_out(acc_sh, out_hbm, c, t)

    return pl.kernel(
        body,
        out_type=jax.ShapeDtypeStruct((2 * N, AW2), jnp.float32),
        mesh=_mesh,
        compiler_params=_sc_params,
        scratch_types=[
            pltpu.VMEM((GE, K), jnp.int32),
            pltpu.VMEM((GE, K), jnp.int32),
            pltpu.VMEM((K, 16), jnp.float32),
            pltpu.VMEM((K, 16), jnp.float32),
            pltpu.VMEM((K, AW2), jnp.float32),
            pltpu.VMEM((K, AW2), jnp.float32),
            pltpu.VMEM((K, AW2), jnp.float32),
            pltpu.VMEM((K, AW2), jnp.float32),
        ] + [pltpu.SemaphoreType.DMA] * 8 + [
            pltpu.VMEM_SHARED((N, AW2), jnp.float32),
        ],
    )


_ps1 = _make_ps1()
_m1 = _make_m1()
_edge2 = _make_edge2()


def kernel(x, edge_index, W1, att_src1, att_dst1, b1, W2, att_src2, att_dst2,
           b2):
    src2 = edge_index[0].reshape(NCH, K)
    dst2 = edge_index[1].reshape(NCH, K)

    rows = jnp.arange(H1)
    heads_of_row = jnp.repeat(jnp.arange(HEADS), HID)
    w_as = jnp.zeros((H1, 16), jnp.float32).at[rows, heads_of_row].set(
        att_src1.reshape(-1))
    w_ad = jnp.zeros((H1, 16), jnp.float32).at[rows, heads_of_row].set(
        att_dst1.reshape(-1))
    w_as2 = jnp.zeros((OUT_CH, 16), jnp.float32).at[:, 0].set(
        att_src2.reshape(-1))
    w_ad2 = jnp.zeros((OUT_CH, 16), jnp.float32).at[:, 0].set(
        att_dst2.reshape(-1))
    cols = jnp.arange(128)
    lh = cols // HID
    p01 = jnp.zeros((16, 128), jnp.float32).at[lh, cols].set(1.0)
    p23 = jnp.zeros((16, 128), jnp.float32).at[lh + 2, cols].set(1.0)
    p45 = jnp.zeros((16, 128), jnp.float32).at[lh + 4, cols].set(1.0)
    zs = jnp.zeros((N, 16), jnp.float32)
    z128 = jnp.zeros((N, 128), jnp.float32)
    z48 = jnp.zeros((N, AW2), jnp.float32)

    h_all, astab, adtab = pl.pallas_call(
        _dense1_body,
        grid=(N // BLK, 3),
        in_specs=[
            pl.BlockSpec((BLK, IN_CH), lambda i, j: (i, 0)),
            pl.BlockSpec((IN_CH, 128), lambda i, j: (0, j)),
            pl.BlockSpec((128, 16), lambda i, j: (j, 0)),
            pl.BlockSpec((128, 16), lambda i, j: (j, 0)),
        ],
        out_specs=[
            pl.BlockSpec((BLK, 128), lambda i, j: (j * (N // BLK) + i, 0)),
            pl.BlockSpec((BLK, 16), lambda i, j: (i, 0)),
            pl.BlockSpec((BLK, 16), lambda i, j: (i, 0)),
        ],
        out_shape=[
            jax.ShapeDtypeStruct((3 * N, 128), jnp.float32),
            jax.ShapeDtypeStruct((N, 16), jnp.float32),
            jax.ShapeDtypeStruct((N, 16), jnp.float32),
        ],
    )(x, W1, w_as, w_ad)

    p3, s1, a45 = _ps1(src2, dst2, astab, adtab, h_all, zs, z128)
    acc1 = _m1(src2, dst2, p3, h_all, z128)

    h2tab, ad2tab = pl.pallas_call(
        _dense2_body,
        grid=(N // BLK,),
        in_specs=[
            pl.BlockSpec((BLK, 128), lambda i: (i, 0)),
            pl.BlockSpec((BLK, 128), lambda i: (i + N // BLK, 0)),
            pl.BlockSpec((BLK, 128), lambda i: (i, 0)),
            pl.BlockSpec((BLK, 128), lambda i: (i + N // BLK, 0)),
            pl.BlockSpec((BLK, 16), lambda i: (i, 0)),
            pl.BlockSpec((BLK, 16), lambda i: (i + N // BLK, 0)),
            pl.BlockSpec((1, H1), lambda i: (0, 0)),
            pl.BlockSpec((H1, OUT_CH), lambda i: (0, 0)),
            pl.BlockSpec((OUT_CH, 16), lambda i: (0, 0)),
            pl.BlockSpec((OUT_CH, 16), lambda i: (0, 0)),
            pl.BlockSpec((16, 128), lambda i: (0, 0)),
            pl.BlockSpec((16, 128), lambda i: (0, 0)),
            pl.BlockSpec((16, 128), lambda i: (0, 0)),
        ],
        out_specs=[
            pl.BlockSpec((BLK, AW2), lambda i: (i, 0)),
            pl.BlockSpec((BLK, 16), lambda i: (i, 0)),
        ],
        out_shape=[
            jax.ShapeDtypeStruct((N, AW2), jnp.float32),
            jax.ShapeDtypeStruct((N, 16), jnp.float32),
        ],
    )(acc1, acc1, a45, a45, s1, s1, b1.reshape(1, H1), W2, w_as2, w_ad2,
      p01, p23, p45)

    acc2 = _edge2(src2, dst2, ad2tab, h2tab, z48)

    out = pl.pallas_call(
        _final_body,
        grid=(N // BLK,),
        in_specs=[
            pl.BlockSpec((BLK, AW2), lambda i: (i, 0)),
            pl.BlockSpec((BLK, AW2), lambda i: (i + N // BLK, 0)),
            pl.BlockSpec((1, OUT_CH), lambda i: (0, 0)),
        ],
        out_specs=pl.BlockSpec((BLK, OUT_CH), lambda i: (i, 0)),
        out_shape=jax.ShapeDtypeStruct((N, OUT_CH), jnp.float32),
    )(acc2, acc2, b2.reshape(1, OUT_CH))
    return out


# speedup vs baseline: 1.0419x; 1.0419x over previous
import jax
import jax.numpy as jnp
from jax import lax
from jax.experimental import pallas as pl
from jax.experimental.pallas import tpu as pltpu
from jax.experimental.pallas import tpu_sc as plsc

N = 10000
E = 320000
IN_CH = 128
HID = 64
HEADS = 6
OUT_CH = 32
H1 = HEADS * HID
AW2 = OUT_CH + 16
K = 80
G = 5
NCH = E // K
BLK = 1000

_mesh = plsc.VectorSubcoreMesh(core_axis_name="c", subcore_axis_name="s")
_sc_params = pltpu.CompilerParams(use_tc_tiling_on_sc=False)


def _dense1_body(x_ref, w1_ref, was_ref, wad_ref, h_ref, as_ref, ad_ref):
    j = pl.program_id(1)
    h = jnp.dot(x_ref[...], w1_ref[...], preferred_element_type=jnp.float32)
    h_ref[...] = h
    pas = jnp.dot(h, was_ref[...], preferred_element_type=jnp.float32)
    pad = jnp.dot(h, wad_ref[...], preferred_element_type=jnp.float32)

    @pl.when(j == 0)
    def _():
        as_ref[...] = pas
        ad_ref[...] = pad

    @pl.when(j > 0)
    def _():
        as_ref[...] += pas
        ad_ref[...] += pad


def _dense2_body(acca_ref, accb_ref, c45a_ref, c45b_ref, sa_ref, sb_ref,
                 b1_ref, w2_ref, was2_ref, wad2_ref, p01_ref, p23_ref,
                 p45_ref, h2_ref, ad2_ref):
    rec = 1.0 / (sa_ref[...] + sb_ref[...] + 1e-16)
    h1 = jnp.concatenate([
        acca_ref[...] * jnp.dot(rec, p01_ref[...],
                                preferred_element_type=jnp.float32),
        accb_ref[...] * jnp.dot(rec, p23_ref[...],
                                preferred_element_type=jnp.float32),
        (c45a_ref[...] + c45b_ref[...]) * jnp.dot(
            rec, p45_ref[...], preferred_element_type=jnp.float32),
    ], axis=1)
    h1 = jnp.maximum(h1 + b1_ref[...], 0.0)
    h2 = jnp.dot(h1, w2_ref[...], preferred_element_type=jnp.float32)
    as2 = jnp.dot(h2, was2_ref[...], preferred_element_type=jnp.float32)
    h2_ref[...] = jnp.concatenate([h2, as2], axis=1)
    ad2_ref[...] = jnp.dot(h2, wad2_ref[...],
                           preferred_element_type=jnp.float32)


def _final_body(acca_ref, accb_ref, b2_ref, o_ref):
    acca = acca_ref[...]
    accb = accb_ref[...]
    num = acca[:, :OUT_CH] + accb[:, :OUT_CH]
    srow = acca[:, OUT_CH:] + accb[:, OUT_CH:]
    lane = lax.broadcasted_iota(jnp.int32, srow.shape, 1)
    s = jnp.sum(jnp.where(lane == 0, srow, 0.0), axis=1, keepdims=True)
    h2 = num / (s + 1e-16) + b2_ref[...]
    m = jnp.max(h2, axis=1, keepdims=True)
    z = h2 - m
    o_ref[...] = z - jnp.log(jnp.sum(jnp.exp(z), axis=1, keepdims=True))


def _zero_shared(z_hbm, acc_sh, t):
    zb = t * 624
    pltpu.sync_copy(z_hbm.at[pl.ds(zb, 624)], acc_sh.at[pl.ds(zb, 624)])

    @pl.when(t == 15)
    def _():
        pltpu.sync_copy(z_hbm.at[pl.ds(9984, 16)], acc_sh.at[pl.ds(9984, 16)])


def _copy_out(acc_sh, out_hbm, c, t):
    zb = t * 624
    pltpu.sync_copy(acc_sh.at[pl.ds(zb, 624)],
                    out_hbm.at[pl.ds(c * N + zb, 624)])

    @pl.when(t == 15)
    def _():
        pltpu.sync_copy(acc_sh.at[pl.ds(9984, 16)],
                        out_hbm.at[pl.ds(c * N + 9984, 16)])


def _shift_idx(grp, delta, out=None):
    dst = grp if out is None else out
    for r in range(grp.shape[0]):
        for s5 in range(K // 16):
            sl = pl.ds(s5 * 16, 16)
            dst[r, sl] = grp[r, sl] + delta


def _make_ps1():
    def body(src2, dst2, astab, adtab, h_all, zs_hbm, z128_hbm,
             p3, s_out, a45_out,
             sgrp, sg2, dgrp, pbuf, a0, a1, b0, b1, h0, h1, s_sh, a45_sh,
             sem_s, sem_d, sem_a0, sem_a1, sem_b0, sem_b1, sem_h0, sem_h1,
             sem_x0, sem_x1, sem_y0, sem_y1):
        c = lax.axis_index("c")
        t = lax.axis_index("s")
        _zero_shared(zs_hbm, s_sh, t)
        _zero_shared(z128_hbm, a45_sh, t)
        plsc.subcore_barrier()
        abufs = (a0, a1)
        bbufs = (b0, b1)
        hbufs = (h0, h1)
        asems = (sem_a0, sem_a1)
        bsems = (sem_b0, sem_b1)
        hsems = (sem_h0, sem_h1)
        xsems = (sem_x0, sem_x1)
        ysems = (sem_y0, sem_y1)
        rbase = c * (NCH // 2) + t * (NCH // 32)
        two_n = jnp.int32(2 * N)

        def group(g, carry):
            r0 = rbase + g * G
            cps = pltpu.async_copy(src2.at[pl.ds(r0, G)], sgrp, sem_s)
            cpd = pltpu.async_copy(dst2.at[pl.ds(r0, G)], dgrp, sem_d)
            cps.wait()
            cpd.wait()
            _shift_idx(sgrp, two_n, out=sg2)
            cpa = [None] * G
            cpb = [None] * G
            cph = [None] * G
            scx = [None] * G
            scy = [None] * G
            cpa[0] = pltpu.async_copy(astab.at[sgrp.at[0]], a0, sem_a0)
            cpb[0] = pltpu.async_copy(adtab.at[dgrp.at[0]], b0, sem_b0)
            cph[0] = pltpu.async_copy(h_all.at[sg2.at[0]], h0, sem_h0)
            for b in range(G):
                if b >= 1:
                    scx[b - 1].wait()
                    scy[b - 1].wait()
                if b + 1 < G:
                    nb = (b + 1) % 2
                    cpa[b + 1] = pltpu.async_copy(
                        astab.at[sgrp.at[b + 1]], abufs[nb], asems[nb])
                    cpb[b + 1] = pltpu.async_copy(
                        adtab.at[dgrp.at[b + 1]], bbufs[nb], bsems[nb])
                    cph[b + 1] = pltpu.async_copy(
                        h_all.at[sg2.at[b + 1]], hbufs[nb], hsems[nb])
                cpa[b].wait()
                cpb[b].wait()
                cph[b].wait()
                asr_v = abufs[b % 2]
                adr_v = bbufs[b % 2]
                h_v = hbufs[b % 2]

                @plsc.parallel_loop(0, K, unroll=2)
                def edge(k):
                    e = asr_v[k] + adr_v[k]
                    p = jnp.exp(jnp.maximum(e, 0.2 * e))
                    pbuf[b, k] = p
                    for j in range(2):
                        pj = jnp.take_along_axis(
                            p, jnp.full((16,), 4 + j, jnp.int32), axis=0,
                            mode="promise_in_bounds")
                        for v in range(4):
                            sl = pl.ds(j * 64 + v * 16, 16)
                            h_v[k, sl] = h_v[k, sl] * pj

                scx[b] = pltpu.async_copy(
                    pbuf.at[b], s_sh.at[dgrp.at[b]], xsems[b % 2], add=True)
                scy[b] = pltpu.async_copy(
                    h_v, a45_sh.at[dgrp.at[b]], ysems[b % 2], add=True)
            scx[G - 1].wait()
            scy[G - 1].wait()
            pltpu.sync_copy(pbuf, p3.at[pl.ds(r0, G)])
            return carry

        lax.fori_loop(0, NCH // 32 // G, group, 0)
        plsc.subcore_barrier()
        _copy_out(s_sh, s_out, c, t)
        _copy_out(a45_sh, a45_out, c, t)

    return pl.kernel(
        body,
        out_type=[
            jax.ShapeDtypeStruct((NCH, K, 16), jnp.float32),
            jax.ShapeDtypeStruct((2 * N, 16), jnp.float32),
            jax.ShapeDtypeStruct((2 * N, 128), jnp.float32),
        ],
        mesh=_mesh,
        compiler_params=_sc_params,
        scratch_types=[
            pltpu.VMEM((G, K), jnp.int32),
            pltpu.VMEM((G, K), jnp.int32),
            pltpu.VMEM((G, K), jnp.int32),
            pltpu.VMEM((G, K, 16), jnp.float32),
            pltpu.VMEM((K, 16), jnp.float32),
            pltpu.VMEM((K, 16), jnp.float32),
            pltpu.VMEM((K, 16), jnp.float32),
            pltpu.VMEM((K, 16), jnp.float32),
            pltpu.VMEM((K, 128), jnp.float32),
            pltpu.VMEM((K, 128), jnp.float32),
            pltpu.VMEM_SHARED((N, 16), jnp.float32),
            pltpu.VMEM_SHARED((N, 128), jnp.float32),
        ] + [pltpu.SemaphoreType.DMA] * 12,
    )


GM = 10


def _make_m1():
    def body(src2, dst2, p3, h_all, z128_hbm, out_hbm,
             sgrp, dgrp, pgrp, h0, h1,
             sem_s, sem_d, sem_p, sem_h0, sem_h1, sem_x0, sem_x1, acc_sh):
        c = lax.axis_index("c")
        t = lax.axis_index("s")
        hoff = c * 2
        _zero_shared(z128_hbm, acc_sh, t)
        plsc.subcore_barrier()
        hbufs = (h0, h1)
        hsems = (sem_h0, sem_h1)
        ssems = (sem_x0, sem_x1)
        rbase = t * (NCH // 16)
        c_n = c * jnp.int32(N)

        def group(g, carry):
            r0 = rbase + g * GM
            cps = pltpu.async_copy(src2.at[pl.ds(r0, GM)], sgrp, sem_s)
            cpd = pltpu.async_copy(dst2.at[pl.ds(r0, GM)], dgrp, sem_d)
            cpp = pltpu.async_copy(p3.at[pl.ds(r0, GM)], pgrp, sem_p)
            cps.wait()
            _shift_idx(sgrp, c_n)
            cph = [None] * GM
            scp = [None] * GM
            cph[0] = pltpu.async_copy(h_all.at[sgrp.at[0]], h0, sem_h0)
            cpp.wait()
            cpd.wait()
            for b in range(GM):
                if b >= 1:
                    scp[b - 1].wait()
                if b + 1 < GM:
                    nb = (b + 1) % 2
                    cph[b + 1] = pltpu.async_copy(
                        h_all.at[sgrp.at[b + 1]], hbufs[nb], hsems[nb])
                cph[b].wait()
                h_v = hbufs[b % 2]

                @plsc.parallel_loop(0, K, unroll=2)
                def edge(k):
                    p = pgrp[b, k]
                    for j in range(2):
                        pj = jnp.take_along_axis(
                            p, jnp.full((16,), hoff + j, jnp.int32), axis=0,
                            mode="promise_in_bounds")
                        for v in range(4):
                            sl = pl.ds(j * 64 + v * 16, 16)
                            h_v[k, sl] = h_v[k, sl] * pj

                scp[b] = pltpu.async_copy(
                    h_v, acc_sh.at[dgrp.at[b]], ssems[b % 2], add=True)
            scp[GM - 1].wait()
            return carry

        lax.fori_loop(0, NCH // 16 // GM, group, 0)
        plsc.subcore_barrier()
        _copy_out(acc_sh, out_hbm, c, t)

    return pl.kernel(
        body,
        out_type=jax.ShapeDtypeStruct((2 * N, 128), jnp.float32),
        mesh=_mesh,
        compiler_params=_sc_params,
        scratch_types=[
            pltpu.VMEM((GM, K), jnp.int32),
            pltpu.VMEM((GM, K), jnp.int32),
            pltpu.VMEM((GM, K, 16), jnp.float32),
            pltpu.VMEM((K, 128), jnp.float32),
            pltpu.VMEM((K, 128), jnp.float32),
            pltpu.SemaphoreType.DMA,
            pltpu.SemaphoreType.DMA,
            pltpu.SemaphoreType.DMA,
            pltpu.SemaphoreType.DMA,
            pltpu.SemaphoreType.DMA,
            pltpu.SemaphoreType.DMA,
            pltpu.SemaphoreType.DMA,
            pltpu.VMEM_SHARED((N, 128), jnp.float32),
        ],
    )


GE = 25


def _make_edge2():
    def body(src2, dst2, adtab, h2_hbm, z48_hbm, out_hbm,
             sgrp, dgrp, b0, b1, h0, h1, m0, m1b,
             sem_s, sem_d, sem_b0, sem_b1, sem_h0, sem_h1, sem_x0, sem_x1,
             acc_sh):
        c = lax.axis_index("c")
        t = lax.axis_index("s")
        _zero_shared(z48_hbm, acc_sh, t)
        plsc.subcore_barrier()
        bbufs = (b0, b1)
        hbufs = (h0, h1)
        bsems = (sem_b0, sem_b1)
        hsems = (sem_h0, sem_h1)
        msgbufs = (m0, m1b)
        xsems = (sem_x0, sem_x1)
        rbase = c * (NCH // 2) + t * (NCH // 32)
        zeros16 = jnp.zeros((16,), jnp.int32)

        def group(g, carry):
            r0 = rbase + g * GE
            cps = pltpu.async_copy(src2.at[pl.ds(r0, GE)], sgrp, sem_s)
            cpd = pltpu.async_copy(dst2.at[pl.ds(r0, GE)], dgrp, sem_d)
            cps.wait()
            cpd.wait()
            cpb = [None] * GE
            cph = [None] * GE
            scp = [None] * GE
            cpb[0] = pltpu.async_copy(adtab.at[dgrp.at[0]], b0, sem_b0)
            cph[0] = pltpu.async_copy(h2_hbm.at[sgrp.at[0]], h0, sem_h0)
            for b in range(GE):
                if b >= 2:
                    scp[b - 2].wait()
                if b + 1 < GE:
                    nb = (b + 1) % 2
                    cpb[b + 1] = pltpu.async_copy(
                        adtab.at[dgrp.at[b + 1]], bbufs[nb], bsems[nb])
                    cph[b + 1] = pltpu.async_copy(
                        h2_hbm.at[sgrp.at[b + 1]], hbufs[nb], hsems[nb])
                cpb[b].wait()
                cph[b].wait()
                adr_v = bbufs[b % 2]
                h_v = hbufs[b % 2]
                msg_v = msgbufs[b % 2]

                @plsc.parallel_loop(0, K, unroll=4)
                def edge(k):
                    e = h_v[k, pl.ds(OUT_CH, 16)] + adr_v[k]
                    p = jnp.exp(jnp.maximum(e, 0.2 * e))
                    msg_v[k, pl.ds(OUT_CH, 16)] = p
                    p0 = jnp.take_along_axis(p, zeros16, axis=0,
                                             mode="promise_in_bounds")
                    msg_v[k, pl.ds(0, 16)] = h_v[k, pl.ds(0, 16)] * p0
                    msg_v[k, pl.ds(16, 16)] = h_v[k, pl.ds(16, 16)] * p0

                scp[b] = pltpu.async_copy(
                    msg_v, acc_sh.at[dgrp.at[b]], xsems[b % 2], add=True)
            scp[GE - 2].wait()
            scp[GE - 1].wait()
            return carry

        lax.fori_loop(0, NCH // 32 // GE, group, 0)
        plsc.subcore_barrier()
        _copy_out(acc_sh, out_hbm, c, t)

    return pl.kernel(
        body,
        out_type=jax.ShapeDtypeStruct((2 * N, AW2), jnp.float32),
        mesh=_mesh,
        compiler_params=_sc_params,
        scratch_types=[
            pltpu.VMEM((GE, K), jnp.int32),
            pltpu.VMEM((GE, K), jnp.int32),
            pltpu.VMEM((K, 16), jnp.float32),
            pltpu.VMEM((K, 16), jnp.float32),
            pltpu.VMEM((K, AW2), jnp.float32),
            pltpu.VMEM((K, AW2), jnp.float32),
            pltpu.VMEM((K, AW2), jnp.float32),
            pltpu.VMEM((K, AW2), jnp.float32),
        ] + [pltpu.SemaphoreType.DMA] * 8 + [
            pltpu.VMEM_SHARED((N, AW2), jnp.float32),
        ],
    )


_ps1 = _make_ps1()
_m1 = _make_m1()
_edge2 = _make_edge2()


def kernel(x, edge_index, W1, att_src1, att_dst1, b1, W2, att_src2, att_dst2,
           b2):
    src2 = edge_index[0].reshape(NCH, K)
    dst2 = edge_index[1].reshape(NCH, K)

    rows = jnp.arange(H1)
    heads_of_row = jnp.repeat(jnp.arange(HEADS), HID)
    w_as = jnp.zeros((H1, 16), jnp.float32).at[rows, heads_of_row].set(
        att_src1.reshape(-1))
    w_ad = jnp.zeros((H1, 16), jnp.float32).at[rows, heads_of_row].set(
        att_dst1.reshape(-1))
    w_as2 = jnp.zeros((OUT_CH, 16), jnp.float32).at[:, 0].set(
        att_src2.reshape(-1))
    w_ad2 = jnp.zeros((OUT_CH, 16), jnp.float32).at[:, 0].set(
        att_dst2.reshape(-1))
    cols = jnp.arange(128)
    lh = cols // HID
    p01 = jnp.zeros((16, 128), jnp.float32).at[lh, cols].set(1.0)
    p23 = jnp.zeros((16, 128), jnp.float32).at[lh + 2, cols].set(1.0)
    p45 = jnp.zeros((16, 128), jnp.float32).at[lh + 4, cols].set(1.0)
    zs = jnp.zeros((N, 16), jnp.float32)
    z128 = jnp.zeros((N, 128), jnp.float32)
    z48 = jnp.zeros((N, AW2), jnp.float32)

    h_all, astab, adtab = pl.pallas_call(
        _dense1_body,
        grid=(N // BLK, 3),
        in_specs=[
            pl.BlockSpec((BLK, IN_CH), lambda i, j: (i, 0)),
            pl.BlockSpec((IN_CH, 128), lambda i, j: (0, j)),
            pl.BlockSpec((128, 16), lambda i, j: (j, 0)),
            pl.BlockSpec((128, 16), lambda i, j: (j, 0)),
        ],
        out_specs=[
            pl.BlockSpec((BLK, 128), lambda i, j: (j * (N // BLK) + i, 0)),
            pl.BlockSpec((BLK, 16), lambda i, j: (i, 0)),
            pl.BlockSpec((BLK, 16), lambda i, j: (i, 0)),
        ],
        out_shape=[
            jax.ShapeDtypeStruct((3 * N, 128), jnp.float32),
            jax.ShapeDtypeStruct((N, 16), jnp.float32),
            jax.ShapeDtypeStruct((N, 16), jnp.float32),
        ],
    )(x, W1, w_as, w_ad)

    p3, s1, a45 = _ps1(src2, dst2, astab, adtab, h_all, zs, z128)
    acc1 = _m1(src2, dst2, p3, h_all, z128)

    h2tab, ad2tab = pl.pallas_call(
        _dense2_body,
        grid=(N // BLK,),
        in_specs=[
            pl.BlockSpec((BLK, 128), lambda i: (i, 0)),
            pl.BlockSpec((BLK, 128), lambda i: (i + N // BLK, 0)),
            pl.BlockSpec((BLK, 128), lambda i: (i, 0)),
            pl.BlockSpec((BLK, 128), lambda i: (i + N // BLK, 0)),
            pl.BlockSpec((BLK, 16), lambda i: (i, 0)),
            pl.BlockSpec((BLK, 16), lambda i: (i + N // BLK, 0)),
            pl.BlockSpec((1, H1), lambda i: (0, 0)),
            pl.BlockSpec((H1, OUT_CH), lambda i: (0, 0)),
            pl.BlockSpec((OUT_CH, 16), lambda i: (0, 0)),
            pl.BlockSpec((OUT_CH, 16), lambda i: (0, 0)),
            pl.BlockSpec((16, 128), lambda i: (0, 0)),
            pl.BlockSpec((16, 128), lambda i: (0, 0)),
            pl.BlockSpec((16, 128), lambda i: (0, 0)),
        ],
        out_specs=[
            pl.BlockSpec((BLK, AW2), lambda i: (i, 0)),
            pl.BlockSpec((BLK, 16), lambda i: (i, 0)),
        ],
        out_shape=[
            jax.ShapeDtypeStruct((N, AW2), jnp.float32),
            jax.ShapeDtypeStruct((N, 16), jnp.float32),
        ],
    )(acc1, acc1, a45, a45, s1, s1, b1.reshape(1, H1), W2, w_as2, w_ad2,
      p01, p23, p45)

    acc2 = _edge2(src2, dst2, ad2tab, h2tab, z48)

    out = pl.pallas_call(
        _final_body,
        grid=(N // BLK,),
        in_specs=[
            pl.BlockSpec((BLK, AW2), lambda i: (i, 0)),
            pl.BlockSpec((BLK, AW2), lambda i: (i + N // BLK, 0)),
            pl.BlockSpec((1, OUT_CH), lambda i: (0, 0)),
        ],
        out_specs=pl.BlockSpec((BLK, OUT_CH), lambda i: (i, 0)),
        out_shape=jax.ShapeDtypeStruct((N, OUT_CH), jnp.float32),
    )(acc2, acc2, b2.reshape(1, OUT_CH))
    return out
